# baseline (speedup 1.0000x reference)
import functools

import jax
import jax.numpy as jnp
from jax import lax
from jax.experimental import pallas as pl
from jax.experimental.pallas import tpu as pltpu
from jax.experimental.pallas import tpu_sc as plsc

L = 16
NC, NS = 2, 16
NW = NC * NS
VP = 2 * L


def _make_kernel(B, SEQ, EMB1):
    rows = B // NW
    chunks = rows // L

    mesh = plsc.VectorSubcoreMesh(core_axis_name="c", subcore_axis_name="s")

    @functools.partial(
        pl.kernel,
        out_type=jax.ShapeDtypeStruct((B,), jnp.float32),
        mesh=mesh,
        compiler_params=pltpu.CompilerParams(needs_layout_passes=False),
        scratch_types=[
            pltpu.VMEM((SEQ * rows,), jnp.int32),
            pltpu.VMEM((EMB1 * VP,), jnp.float32),
            pltpu.VMEM((EMB1 * L,), jnp.float32),
            pltpu.VMEM((VP,), jnp.float32),
            pltpu.VMEM((rows,), jnp.float32),
            pltpu.SemaphoreType.DMA,
            pltpu.SemaphoreType.DMA,
        ],
    )
    def sc_kernel(tok_hbm, tt_hbm, wb_hbm, out_hbm,
                  tok_v, tt_v, wb_v, lut_v, out_v, tsem, wsem):
        wid = lax.axis_index("s") * NC + lax.axis_index("c")

        tok_cp = pltpu.async_copy(tok_hbm.at[wid], tok_v, tsem)
        tt_cp = pltpu.async_copy(tt_hbm, tt_v, wsem)
        wb_cp = pltpu.async_copy(wb_hbm, wb_v, wsem)
        tt_cp.wait()
        wb_cp.wait()

        a0 = jnp.zeros((L,), jnp.float32)
        a1 = jnp.zeros((L,), jnp.float32)
        for d in range(EMB1):
            w = wb_v[pl.ds(d * L, L)]
            a0 = a0 + tt_v[pl.ds(d * VP, L)] * w
            a1 = a1 + tt_v[pl.ds(d * VP + L, L)] * w
        inv_seq = jnp.float32(1.0 / SEQ)
        lut_v[pl.ds(0, L)] = a0 * inv_seq
        lut_v[pl.ds(L, L)] = a1 * inv_seq

        tok_cp.wait()

        pos0 = lax.iota(jnp.int32, L) * SEQ
        for j in range(chunks):
            g = None
            for s in range(SEQ):
                t = plsc.load_gather(tok_v, [pos0 + (j * L * SEQ + s)])
                gs = plsc.load_gather(lut_v, [t])
                g = gs if g is None else g + gs
            out_v[pl.ds(j * L, L)] = 1.0 / (1.0 + jnp.exp(-g))

        pltpu.sync_copy(out_v, out_hbm.at[pl.ds(wid * rows, rows)])

    return sc_kernel


def kernel(tokens, table, W, b):
    B, SEQ = tokens.shape
    VOCAB, EMB = table.shape
    rows = B // NW

    tok_w = tokens.reshape(NW, rows * SEQ)
    tt = jnp.pad(table.T, ((0, 1), (0, VP - VOCAB)),
                 constant_values=1.0).reshape(-1)
    wb = jnp.concatenate([W.reshape(EMB), b]).reshape(EMB + 1, 1)
    wb = jnp.broadcast_to(wb, (EMB + 1, L)).reshape(-1)

    out = _make_kernel(B, SEQ, EMB + 1)(tok_w, tt, wb)
    return out.reshape(B, 1)


# speedup vs baseline: 1.0027x; 1.0027x over previous
import functools

import jax
import jax.numpy as jnp
from jax import lax
from jax.experimental import pallas as pl
from jax.experimental.pallas import tpu as pltpu
from jax.experimental.pallas import tpu_sc as plsc

L = 16
NC, NS = 2, 16
NW = NC * NS
VP = 2 * L


def _make_kernel(B, SEQ, EMB1):
    rows = B // NW
    chunks = rows // L

    mesh = plsc.VectorSubcoreMesh(core_axis_name="c", subcore_axis_name="s")

    @functools.partial(
        pl.kernel,
        out_type=jax.ShapeDtypeStruct((B,), jnp.float32),
        mesh=mesh,
        compiler_params=pltpu.CompilerParams(needs_layout_passes=False),
        scratch_types=[
            pltpu.VMEM((SEQ * rows,), jnp.int32),
            pltpu.VMEM((EMB1 * VP,), jnp.float32),
            pltpu.VMEM((EMB1 * L,), jnp.float32),
            pltpu.VMEM((VP,), jnp.float32),
            pltpu.VMEM((rows,), jnp.float32),
            pltpu.SemaphoreType.DMA,
            pltpu.SemaphoreType.DMA,
        ],
    )
    def sc_kernel(tok_hbm, tt_hbm, wb_hbm, out_hbm,
                  tok_v, tt_v, wb_v, lut_v, out_v, tsem, wsem):
        wid = lax.axis_index("s") * NC + lax.axis_index("c")

        tok_cp = pltpu.async_copy(tok_hbm.at[wid], tok_v, tsem)
        tt_cp = pltpu.async_copy(tt_hbm, tt_v, wsem)
        wb_cp = pltpu.async_copy(wb_hbm, wb_v, wsem)
        tt_cp.wait()
        wb_cp.wait()

        a0 = jnp.zeros((L,), jnp.float32)
        a1 = jnp.zeros((L,), jnp.float32)
        for d in range(EMB1):
            w = wb_v[pl.ds(d * L, L)]
            a0 = a0 + tt_v[pl.ds(d * VP, L)] * w
            a1 = a1 + tt_v[pl.ds(d * VP + L, L)] * w
        inv_seq = jnp.float32(1.0 / SEQ)
        lut_v[pl.ds(0, L)] = a0 * inv_seq
        lut_v[pl.ds(L, L)] = a1 * inv_seq

        tok_cp.wait()

        viota_seq = lax.iota(jnp.int32, L) * SEQ
        for j in range(chunks):
            g = None
            for s in range(SEQ):
                t = plsc.load_gather(tok_v, [viota_seq + (j * L * SEQ + s)])
                gs = plsc.load_gather(lut_v, [t])
                g = gs if g is None else g + gs
            out_v[pl.ds(j * L, L)] = 1.0 / (1.0 + jnp.exp(-g))

        pltpu.sync_copy(out_v, out_hbm.at[pl.ds(wid * rows, rows)])

    return sc_kernel


def kernel(tokens, table, W, b):
    B, SEQ = tokens.shape
    VOCAB, EMB = table.shape
    rows = B // NW

    tok_w = tokens.reshape(NW, rows * SEQ)
    tt = jnp.pad(table.T, ((0, 1), (0, VP - VOCAB)),
                 constant_values=1.0).reshape(-1)
    wb = jnp.concatenate([W.reshape(EMB), b]).reshape(EMB + 1, 1)
    wb = jnp.broadcast_to(wb, (EMB + 1, L)).reshape(-1)

    out = _make_kernel(B, SEQ, EMB + 1)(tok_w, tt, wb)
    return out.reshape(B, 1)


# speedup vs baseline: 1.0605x; 1.0577x over previous
import functools

import jax
import jax.numpy as jnp
from jax import lax
from jax.experimental import pallas as pl
from jax.experimental.pallas import tpu as pltpu
from jax.experimental.pallas import tpu_sc as plsc

L = 16
NC, NS = 2, 16
NW = NC * NS
VP = 2 * L


def _make_kernel(B, SEQ, EMB1):
    rows = B // NW
    chunks = rows // L

    mesh = plsc.VectorSubcoreMesh(core_axis_name="c", subcore_axis_name="s")

    @functools.partial(
        pl.kernel,
        out_type=jax.ShapeDtypeStruct((B,), jnp.float32),
        mesh=mesh,
        compiler_params=pltpu.CompilerParams(needs_layout_passes=False),
        scratch_types=[
            pltpu.VMEM((rows, SEQ), jnp.int32),
            pltpu.VMEM((EMB1 * VP,), jnp.float32),
            pltpu.VMEM((EMB1 * L,), jnp.float32),
            pltpu.VMEM((VP,), jnp.float32),
            pltpu.VMEM((rows,), jnp.float32),
            pltpu.SemaphoreType.DMA,
            pltpu.SemaphoreType.DMA,
        ],
    )
    def sc_kernel(tok_hbm, tt_hbm, wb_hbm, out_hbm,
                  tok_v, tt_v, wb_v, lut_v, out_v, tsem, wsem):
        wid = lax.axis_index("s") * NC + lax.axis_index("c")

        tok_cp = pltpu.async_copy(tok_hbm.at[pl.ds(wid * rows, rows)], tok_v, tsem)
        tt_cp = pltpu.async_copy(tt_hbm, tt_v, wsem)
        wb_cp = pltpu.async_copy(wb_hbm, wb_v, wsem)
        tt_cp.wait()
        wb_cp.wait()

        a0 = jnp.zeros((L,), jnp.float32)
        a1 = jnp.zeros((L,), jnp.float32)
        for d in range(EMB1):
            w = wb_v[pl.ds(d * L, L)]
            a0 = a0 + tt_v[pl.ds(d * VP, L)] * w
            a1 = a1 + tt_v[pl.ds(d * VP + L, L)] * w
        inv_seq = jnp.float32(1.0 / SEQ)
        lut_v[pl.ds(0, L)] = a0 * inv_seq
        lut_v[pl.ds(L, L)] = a1 * inv_seq

        tok_cp.wait()

        viota = lax.iota(jnp.int32, L)
        for j in range(chunks):
            g = None
            rr = viota + (j * L)
            for s in range(SEQ):
                t = plsc.load_gather(tok_v, [rr, jnp.full((L,), s, jnp.int32)])
                gs = plsc.load_gather(lut_v, [t])
                g = gs if g is None else g + gs
            out_v[pl.ds(j * L, L)] = 1.0 / (1.0 + jnp.exp(-g))

        pltpu.sync_copy(out_v, out_hbm.at[pl.ds(wid * rows, rows)])

    return sc_kernel


def kernel(tokens, table, W, b):
    B, SEQ = tokens.shape
    VOCAB, EMB = table.shape
    rows = B // NW

    tt = jnp.pad(table.T, ((0, 1), (0, VP - VOCAB)),
                 constant_values=1.0).reshape(-1)
    wb = jnp.concatenate([W.reshape(EMB), b]).reshape(EMB + 1, 1)
    wb = jnp.broadcast_to(wb, (EMB + 1, L)).reshape(-1)

    out = _make_kernel(B, SEQ, EMB + 1)(tokens, tt, wb)
    return out.reshape(B, 1)


# speedup vs baseline: 1.4368x; 1.3548x over previous
import functools

import jax
import jax.numpy as jnp
from jax import lax
from jax.experimental import pallas as pl
from jax.experimental.pallas import tpu as pltpu
from jax.experimental.pallas import tpu_sc as plsc

L = 16
NC, NS = 2, 16
NW = NC * NS
VP = 2 * L


def _make_kernel(B, SEQ, EMB1):
    rows = B // NW
    chunks = rows // L

    mesh = plsc.VectorSubcoreMesh(core_axis_name="c", subcore_axis_name="s")

    @functools.partial(
        pl.kernel,
        out_type=jax.ShapeDtypeStruct((B,), jnp.float32),
        mesh=mesh,
        compiler_params=pltpu.CompilerParams(needs_layout_passes=False),
        scratch_types=[
            pltpu.VMEM((SEQ * rows,), jnp.int32),
            pltpu.VMEM((EMB1 * VP,), jnp.float32),
            pltpu.VMEM((EMB1 * L,), jnp.float32),
            pltpu.VMEM((VP,), jnp.float32),
            pltpu.VMEM((rows,), jnp.float32),
            pltpu.SemaphoreType.DMA,
            pltpu.SemaphoreType.DMA,
        ],
    )
    def sc_kernel(tok_hbm, tt_hbm, wb_hbm, out_hbm,
                  tok_v, tt_v, wb_v, lut_v, out_v, tsem, wsem):
        wid = lax.axis_index("s") * NC + lax.axis_index("c")

        tok_cp = pltpu.async_copy(tok_hbm.at[wid], tok_v, tsem)
        tt_cp = pltpu.async_copy(tt_hbm, tt_v, wsem)
        wb_cp = pltpu.async_copy(wb_hbm, wb_v, wsem)
        tt_cp.wait()
        wb_cp.wait()

        def lut_body(d, acc):
            a0, a1 = acc
            w = wb_v[pl.ds(d * L, L)]
            return (a0 + tt_v[pl.ds(d * VP, L)] * w,
                    a1 + tt_v[pl.ds(d * VP + L, L)] * w)

        a0, a1 = lax.fori_loop(
            0, EMB1, lut_body,
            (jnp.zeros((L,), jnp.float32), jnp.zeros((L,), jnp.float32)))
        inv_seq = jnp.float32(1.0 / SEQ)
        lut_v[pl.ds(0, L)] = a0 * inv_seq
        lut_v[pl.ds(L, L)] = a1 * inv_seq

        tok_cp.wait()

        def row_body(j, carry):
            g = None
            for s in range(SEQ):
                t = tok_v[pl.ds(s * rows + j * L, L)]
                gs = plsc.load_gather(lut_v, [t])
                g = gs if g is None else g + gs
            out_v[pl.ds(j * L, L)] = 1.0 / (1.0 + jnp.exp(-g))
            return carry

        lax.fori_loop(0, chunks, row_body, jnp.int32(0))

        pltpu.sync_copy(out_v, out_hbm.at[pl.ds(wid * rows, rows)])

    return sc_kernel


def kernel(tokens, table, W, b):
    B, SEQ = tokens.shape
    VOCAB, EMB = table.shape
    rows = B // NW

    tok_w = tokens.reshape(NW, rows, SEQ).transpose(0, 2, 1).reshape(NW, SEQ * rows)
    tt = jnp.pad(table.T, ((0, 1), (0, VP - VOCAB)),
                 constant_values=1.0).reshape(-1)
    wb = jnp.concatenate([W.reshape(EMB), b]).reshape(EMB + 1, 1)
    wb = jnp.broadcast_to(wb, (EMB + 1, L)).reshape(-1)

    out = _make_kernel(B, SEQ, EMB + 1)(tok_w, tt, wb)
    return out.reshape(B, 1)
